# baseline (speedup 1.0000x reference)
import functools

import jax
import jax.numpy as jnp
from jax import lax
from jax.experimental import pallas as pl
from jax.experimental.pallas import tpu as pltpu
from jax.experimental.pallas import tpu_sc as plsc

N_NODES = 10000
N_EDGES = 160000
D_IN = 256
D_OUT = 256

NC = 2
NS = 16
L = 16

H = D_IN // 2
C = 80
NB = 4
NI = 8
GD = 2
DI = 4
K = -(-(-(-N_EDGES // (NS * C))) // NI) * NI
EPT = K * C
E_PAD = EPT * NS
N_ACC = N_NODES + 8
N_CNT = 10240
CSLC = N_CNT // NS
RPT = 624
RCHUNK = 48
R_TAIL = N_NODES - RPT * NS


def _sc_body(edges_hbm, xlo, xhi,
             sum_lo, sum_hi, cnt_out,
             i0, i1, i2, i3, i4, i5, i6, i7,
             r0, r1, r2, r3, zer_v, ones_v,
             acc_sh, cnt_sh,
             sg0, sg1, sg2, sg3, ss0, ss1, ss2, ss3,
             si0, si1, si2, si3, si4, si5, si6, si7,
             sh0, sh1, sh2, sh3, sh4, sh5, sh6, sh7):
    cid = lax.axis_index("c")
    sid = lax.axis_index("s")
    idx = (i0, i1, i2, i3, i4, i5, i6, i7)
    rows = (r0, r1, r2, r3)
    sem_g = (sg0, sg1, sg2, sg3)
    sem_s = (ss0, ss1, ss2, ss3)
    sem_i = (si0, si1, si2, si3, si4, si5, si6, si7)
    sem_h = (sh0, sh1, sh2, sh3, sh4, sh5, sh6, sh7)

    def zrows(i, _):
        def zcol(j, _):
            r0[i, pl.ds(j * L, L)] = jnp.zeros((L,), jnp.float32)
            return 0

        lax.fori_loop(0, H // L, zcol, 0)
        return 0

    lax.fori_loop(0, RCHUNK, zrows, 0)

    def zbuf(i, _):
        zer_v[pl.ds(i * L, L)] = jnp.zeros((L,), jnp.float32)
        return 0

    lax.fori_loop(0, CSLC // L, zbuf, 0)

    def fones(i, _):
        ones_v[pl.ds(i * L, L)] = jnp.ones((L,), jnp.float32)
        return 0

    lax.fori_loop(0, C // L, fones, 0)

    pltpu.sync_copy(zer_v, cnt_sh.at[pl.ds(sid * CSLC, CSLC)])
    base = sid * RPT
    for j in range(RPT // RCHUNK):
        r = base + j * RCHUNK
        pltpu.sync_copy(r0.at[pl.ds(0, RCHUNK)], acc_sh.at[pl.ds(r, RCHUNK)])

    @pl.when(sid == 0)
    def _():
        nt = N_ACC - RPT * NS
        pltpu.sync_copy(r0.at[pl.ds(0, nt)], acc_sh.at[pl.ds(RPT * NS, nt)])

    plsc.subcore_barrier()

    def idx_start(g, q):
        pltpu.async_copy(edges_hbm.at[:, sid, g], idx[q], sem_i[q])

    def idx_wait(g, q):
        pltpu.make_async_copy(edges_hbm.at[:, sid, g], idx[q], sem_i[q]).wait()

    def gather_start(g, b, q):
        @pl.when(cid == 0)
        def _():
            pltpu.async_copy(xlo.at[idx[q].at[0]], rows[b], sem_g[b])

        @pl.when(cid == 1)
        def _():
            pltpu.async_copy(xhi.at[idx[q].at[0]], rows[b], sem_g[b])

    def gather_wait(g, b, q):
        @pl.when(cid == 0)
        def _():
            pltpu.make_async_copy(xlo.at[idx[q].at[0]], rows[b], sem_g[b]).wait()

        @pl.when(cid == 1)
        def _():
            pltpu.make_async_copy(xhi.at[idx[q].at[0]], rows[b], sem_g[b]).wait()

    def scatter_start(b, q):
        pltpu.async_copy(rows[b], acc_sh.at[idx[q].at[1]], sem_s[b], add=True)

    def scatter_wait(b, q):
        pltpu.make_async_copy(rows[b], acc_sh.at[idx[q].at[1]], sem_s[b]).wait()

    def hist_start(q):
        pltpu.async_copy(ones_v, cnt_sh.at[idx[q].at[1]], sem_h[q], add=True)

    def hist_wait(q):
        pltpu.make_async_copy(ones_v, cnt_sh.at[idx[q].at[1]], sem_h[q]).wait()

    for q in range(DI):
        idx_start(q, q)
    for g in range(GD):
        idx_wait(g, g)
        gather_start(g, g, g)

    def step(i, _):
        for u in range(NI):
            g = i * NI + u
            b = u % NB
            q = u % NI

            gather_wait(g, b, q)

            @pl.when(cid == 0)
            def _():
                pltpu.sync_copy(ones_v, cnt_sh.at[idx[q].at[1]], add=True)

            scatter_start(b, q)

            jx = g + DI
            qj = (u + DI) % NI

            @pl.when(jx < K)
            def _():
                idx_start(jx, qj)

            h = g + GD
            bh = (u + GD) % NB
            qh = (u + GD) % NI

            @pl.when(h < K)
            def _():
                @pl.when(h - NB >= 0)
                def _():
                    scatter_wait(bh, qh)

                idx_wait(h, qh)
                gather_start(h, bh, qh)

        return 0

    lax.fori_loop(0, K // NI, step, 0)

    for u in range(NB):
        scatter_wait((K - NB + u) % NB, (K - NB + u) % NI)

    plsc.subcore_barrier()

    @pl.when(cid == 0)
    def _():
        pltpu.sync_copy(cnt_sh.at[pl.ds(sid * CSLC, CSLC)], zer_v)
        pltpu.sync_copy(zer_v, cnt_out.at[pl.ds(sid * CSLC, CSLC)])

    def copy_out(r, nrows):
        pltpu.sync_copy(acc_sh.at[pl.ds(r, nrows)], r0.at[pl.ds(0, nrows)])

        @pl.when(cid == 0)
        def _():
            pltpu.sync_copy(r0.at[pl.ds(0, nrows)], sum_lo.at[pl.ds(r, nrows)])

        @pl.when(cid == 1)
        def _():
            pltpu.sync_copy(r0.at[pl.ds(0, nrows)], sum_hi.at[pl.ds(r, nrows)])

    for j in range(RPT // RCHUNK):
        copy_out(base + j * RCHUNK, RCHUNK)

    @pl.when(sid == 0)
    def _():
        copy_out(RPT * NS, R_TAIL)


_sc_segment_sum = functools.partial(
    pl.kernel,
    out_type=[
        jax.ShapeDtypeStruct((N_NODES, H), jnp.float32),
        jax.ShapeDtypeStruct((N_NODES, H), jnp.float32),
        jax.ShapeDtypeStruct((N_CNT,), jnp.float32),
    ],
    mesh=plsc.VectorSubcoreMesh(
        core_axis_name="c", subcore_axis_name="s", num_cores=NC, num_subcores=NS
    ),
    scratch_types=(
        [pltpu.VMEM((2, C), jnp.int32)] * NI
        + [pltpu.VMEM((C, H), jnp.float32)] * NB
        + [
            pltpu.VMEM((CSLC,), jnp.float32),
            pltpu.VMEM((C,), jnp.float32),
            pltpu.VMEM_SHARED((N_ACC, H), jnp.float32),
            pltpu.VMEM_SHARED((N_CNT,), jnp.float32),
        ]
        + [pltpu.SemaphoreType.DMA] * (NB + NB + NI + NI)
    ),
)(_sc_body)


R_BLK = 1024


def _tc_body(x_ref, slo_ref, shi_ref, cnt_ref, wl_ref, wr_ref, b_ref, o_ref):
    cnt_row = jnp.reshape(cnt_ref[...], (1, R_BLK))
    cnt_col = lax.dot_general(
        cnt_row, jnp.ones((1, 1), jnp.float32),
        (((0,), (0,)), ((), ())), preferred_element_type=jnp.float32,
        precision=lax.Precision.HIGHEST)
    inv = 1.0 / jnp.maximum(cnt_col, 1.0)
    ml = slo_ref[...] * inv
    mh = shi_ref[...] * inv
    acc = jnp.dot(ml, wl_ref[0:H, :], preferred_element_type=jnp.float32,
                  precision=lax.Precision.HIGHEST)
    acc += jnp.dot(mh, wl_ref[H:D_IN, :], preferred_element_type=jnp.float32,
                   precision=lax.Precision.HIGHEST)
    acc += jnp.dot(x_ref[...], wr_ref[...], preferred_element_type=jnp.float32,
                   precision=lax.Precision.HIGHEST)
    o_ref[...] = acc + b_ref[...]


def _tc_combine(x, sum_lo, sum_hi, cnt, W_l, W_r, b2):
    grid = -(-N_NODES // R_BLK)
    return pl.pallas_call(
        _tc_body,
        grid=(grid,),
        in_specs=[
            pl.BlockSpec((R_BLK, D_IN), lambda i: (i, 0)),
            pl.BlockSpec((R_BLK, H), lambda i: (i, 0)),
            pl.BlockSpec((R_BLK, H), lambda i: (i, 0)),
            pl.BlockSpec((R_BLK,), lambda i: (i,)),
            pl.BlockSpec((D_IN, D_OUT), lambda i: (0, 0)),
            pl.BlockSpec((D_IN, D_OUT), lambda i: (0, 0)),
            pl.BlockSpec((1, D_OUT), lambda i: (0, 0)),
        ],
        out_specs=pl.BlockSpec((R_BLK, D_OUT), lambda i: (i, 0)),
        out_shape=jax.ShapeDtypeStruct((N_NODES, D_OUT), jnp.float32),
    )(x, sum_lo, sum_hi, cnt, W_l, W_r, b2)


@jax.jit
def kernel(x, edge_index, W_l, W_r, b):
    ei = edge_index.astype(jnp.int32)
    pad = E_PAD - N_EDGES
    pad_blk = jnp.stack([
        jnp.zeros((pad,), jnp.int32),
        jnp.full((pad,), N_NODES, jnp.int32),
    ])
    edges = jnp.concatenate([ei, pad_blk], axis=1).reshape(2, NS, K, C)
    xlo = x[:, :H]
    xhi = x[:, H:]
    sum_lo, sum_hi, cnt = _sc_segment_sum(edges, xlo, xhi)
    return _tc_combine(x, sum_lo, sum_hi, cnt, W_l, W_r, b.reshape(1, D_OUT))


# speedup vs baseline: 1.0280x; 1.0280x over previous
import functools

import jax
import jax.numpy as jnp
from jax import lax
from jax.experimental import pallas as pl
from jax.experimental.pallas import tpu as pltpu
from jax.experimental.pallas import tpu_sc as plsc

N_NODES = 10000
N_EDGES = 160000
D_IN = 256
D_OUT = 256

NC = 2
NS = 16
L = 16

H = D_IN // 2
C = 80
NB = 4
NI = 8
GD = 3
DI = 4
K = -(-(-(-N_EDGES // (NS * C))) // NI) * NI
EPT = K * C
E_PAD = EPT * NS
N_ACC = N_NODES + 8
N_CNT = 10240
CSLC = N_CNT // NS
RPT = 624
RCHUNK = 48
R_TAIL = N_NODES - RPT * NS


def _sc_body(edges_hbm, xlo, xhi,
             sum_lo, sum_hi, cnt_out,
             i0, i1, i2, i3, i4, i5, i6, i7,
             r0, r1, r2, r3, zer_v, ones_v,
             acc_sh, cnt_sh,
             sg0, sg1, sg2, sg3, ss0, ss1, ss2, ss3,
             si0, si1, si2, si3, si4, si5, si6, si7,
             sh0, sh1, sh2, sh3, sh4, sh5, sh6, sh7):
    cid = lax.axis_index("c")
    sid = lax.axis_index("s")
    idx = (i0, i1, i2, i3, i4, i5, i6, i7)
    rows = (r0, r1, r2, r3)
    sem_g = (sg0, sg1, sg2, sg3)
    sem_s = (ss0, ss1, ss2, ss3)
    sem_i = (si0, si1, si2, si3, si4, si5, si6, si7)
    sem_h = (sh0, sh1, sh2, sh3, sh4, sh5, sh6, sh7)

    def zrows(i, _):
        def zcol(j, _):
            r0[i, pl.ds(j * L, L)] = jnp.zeros((L,), jnp.float32)
            return 0

        lax.fori_loop(0, H // L, zcol, 0)
        return 0

    lax.fori_loop(0, RCHUNK, zrows, 0)

    def zbuf(i, _):
        zer_v[pl.ds(i * L, L)] = jnp.zeros((L,), jnp.float32)
        return 0

    lax.fori_loop(0, CSLC // L, zbuf, 0)

    def fones(i, _):
        ones_v[pl.ds(i * L, L)] = jnp.ones((L,), jnp.float32)
        return 0

    lax.fori_loop(0, C // L, fones, 0)

    pltpu.sync_copy(zer_v, cnt_sh.at[pl.ds(sid * CSLC, CSLC)])
    base = sid * RPT
    for j in range(RPT // RCHUNK):
        r = base + j * RCHUNK
        pltpu.sync_copy(r0.at[pl.ds(0, RCHUNK)], acc_sh.at[pl.ds(r, RCHUNK)])

    @pl.when(sid == 0)
    def _():
        nt = N_ACC - RPT * NS
        pltpu.sync_copy(r0.at[pl.ds(0, nt)], acc_sh.at[pl.ds(RPT * NS, nt)])

    plsc.subcore_barrier()

    def idx_start(g, q):
        pltpu.async_copy(edges_hbm.at[:, sid, g], idx[q], sem_i[q])

    def idx_wait(g, q):
        pltpu.make_async_copy(edges_hbm.at[:, sid, g], idx[q], sem_i[q]).wait()

    def gather_start(g, b, q):
        @pl.when(cid == 0)
        def _():
            pltpu.async_copy(xlo.at[idx[q].at[0]], rows[b], sem_g[b])

        @pl.when(cid == 1)
        def _():
            pltpu.async_copy(xhi.at[idx[q].at[0]], rows[b], sem_g[b])

    def gather_wait(g, b, q):
        @pl.when(cid == 0)
        def _():
            pltpu.make_async_copy(xlo.at[idx[q].at[0]], rows[b], sem_g[b]).wait()

        @pl.when(cid == 1)
        def _():
            pltpu.make_async_copy(xhi.at[idx[q].at[0]], rows[b], sem_g[b]).wait()

    def scatter_start(b, q):
        pltpu.async_copy(rows[b], acc_sh.at[idx[q].at[1]], sem_s[b], add=True)

    def scatter_wait(b, q):
        pltpu.make_async_copy(rows[b], acc_sh.at[idx[q].at[1]], sem_s[b]).wait()

    def hist_start(q):
        pltpu.async_copy(ones_v, cnt_sh.at[idx[q].at[1]], sem_h[q], add=True)

    def hist_wait(q):
        pltpu.make_async_copy(ones_v, cnt_sh.at[idx[q].at[1]], sem_h[q]).wait()

    for q in range(DI):
        idx_start(q, q)
    for g in range(GD):
        idx_wait(g, g)
        gather_start(g, g, g)

    def step(i, _):
        for u in range(NI):
            g = i * NI + u
            b = u % NB
            q = u % NI

            gather_wait(g, b, q)

            @pl.when(cid == 0)
            def _():
                pltpu.sync_copy(ones_v, cnt_sh.at[idx[q].at[1]], add=True)

            scatter_start(b, q)

            jx = g + DI
            qj = (u + DI) % NI

            @pl.when(jx < K)
            def _():
                idx_start(jx, qj)

            h = g + GD
            bh = (u + GD) % NB
            qh = (u + GD) % NI

            @pl.when(h < K)
            def _():
                @pl.when(h - NB >= 0)
                def _():
                    scatter_wait(bh, qh)

                idx_wait(h, qh)
                gather_start(h, bh, qh)

        return 0

    lax.fori_loop(0, K // NI, step, 0)

    for u in range(NB):
        scatter_wait((K - NB + u) % NB, (K - NB + u) % NI)

    plsc.subcore_barrier()

    @pl.when(cid == 0)
    def _():
        pltpu.sync_copy(cnt_sh.at[pl.ds(sid * CSLC, CSLC)], zer_v)
        pltpu.sync_copy(zer_v, cnt_out.at[pl.ds(sid * CSLC, CSLC)])

    def copy_out(r, nrows):
        pltpu.sync_copy(acc_sh.at[pl.ds(r, nrows)], r0.at[pl.ds(0, nrows)])

        @pl.when(cid == 0)
        def _():
            pltpu.sync_copy(r0.at[pl.ds(0, nrows)], sum_lo.at[pl.ds(r, nrows)])

        @pl.when(cid == 1)
        def _():
            pltpu.sync_copy(r0.at[pl.ds(0, nrows)], sum_hi.at[pl.ds(r, nrows)])

    for j in range(RPT // RCHUNK):
        copy_out(base + j * RCHUNK, RCHUNK)

    @pl.when(sid == 0)
    def _():
        copy_out(RPT * NS, R_TAIL)


_sc_segment_sum = functools.partial(
    pl.kernel,
    out_type=[
        jax.ShapeDtypeStruct((N_NODES, H), jnp.float32),
        jax.ShapeDtypeStruct((N_NODES, H), jnp.float32),
        jax.ShapeDtypeStruct((N_CNT,), jnp.float32),
    ],
    mesh=plsc.VectorSubcoreMesh(
        core_axis_name="c", subcore_axis_name="s", num_cores=NC, num_subcores=NS
    ),
    scratch_types=(
        [pltpu.VMEM((2, C), jnp.int32)] * NI
        + [pltpu.VMEM((C, H), jnp.float32)] * NB
        + [
            pltpu.VMEM((CSLC,), jnp.float32),
            pltpu.VMEM((C,), jnp.float32),
            pltpu.VMEM_SHARED((N_ACC, H), jnp.float32),
            pltpu.VMEM_SHARED((N_CNT,), jnp.float32),
        ]
        + [pltpu.SemaphoreType.DMA] * (NB + NB + NI + NI)
    ),
)(_sc_body)


R_BLK = 1024


def _tc_body(x_ref, slo_ref, shi_ref, cnt_ref, wl_ref, wr_ref, b_ref, o_ref):
    cnt_row = jnp.reshape(cnt_ref[...], (1, R_BLK))
    cnt_col = lax.dot_general(
        cnt_row, jnp.ones((1, 1), jnp.float32),
        (((0,), (0,)), ((), ())), preferred_element_type=jnp.float32,
        precision=lax.Precision.HIGHEST)
    inv = 1.0 / jnp.maximum(cnt_col, 1.0)
    ml = slo_ref[...] * inv
    mh = shi_ref[...] * inv
    acc = jnp.dot(ml, wl_ref[0:H, :], preferred_element_type=jnp.float32,
                  precision=lax.Precision.HIGHEST)
    acc += jnp.dot(mh, wl_ref[H:D_IN, :], preferred_element_type=jnp.float32,
                   precision=lax.Precision.HIGHEST)
    acc += jnp.dot(x_ref[...], wr_ref[...], preferred_element_type=jnp.float32,
                   precision=lax.Precision.HIGHEST)
    o_ref[...] = acc + b_ref[...]


def _tc_combine(x, sum_lo, sum_hi, cnt, W_l, W_r, b2):
    grid = -(-N_NODES // R_BLK)
    return pl.pallas_call(
        _tc_body,
        grid=(grid,),
        in_specs=[
            pl.BlockSpec((R_BLK, D_IN), lambda i: (i, 0)),
            pl.BlockSpec((R_BLK, H), lambda i: (i, 0)),
            pl.BlockSpec((R_BLK, H), lambda i: (i, 0)),
            pl.BlockSpec((R_BLK,), lambda i: (i,)),
            pl.BlockSpec((D_IN, D_OUT), lambda i: (0, 0)),
            pl.BlockSpec((D_IN, D_OUT), lambda i: (0, 0)),
            pl.BlockSpec((1, D_OUT), lambda i: (0, 0)),
        ],
        out_specs=pl.BlockSpec((R_BLK, D_OUT), lambda i: (i, 0)),
        out_shape=jax.ShapeDtypeStruct((N_NODES, D_OUT), jnp.float32),
    )(x, sum_lo, sum_hi, cnt, W_l, W_r, b2)


@jax.jit
def kernel(x, edge_index, W_l, W_r, b):
    ei = edge_index.astype(jnp.int32)
    pad = E_PAD - N_EDGES
    pad_blk = jnp.stack([
        jnp.zeros((pad,), jnp.int32),
        jnp.full((pad,), N_NODES, jnp.int32),
    ])
    edges = jnp.concatenate([ei, pad_blk], axis=1).reshape(2, NS, K, C)
    xlo = x[:, :H]
    xhi = x[:, H:]
    sum_lo, sum_hi, cnt = _sc_segment_sum(edges, xlo, xhi)
    return _tc_combine(x, sum_lo, sum_hi, cnt, W_l, W_r, b.reshape(1, D_OUT))
